# Initial kernel scaffold; baseline (speedup 1.0000x reference)
import jax
import jax.numpy as jnp
from jax.experimental import pallas as pl


def kernel(num_0, num_1, num_2, num_3, num_4, num_5, num_6, num_7, num_8, num_9, num_10, num_11, num_12, cat_0, cat_1, cat_2, cat_3, cat_4, cat_5, cat_6, cat_7, cat_8, cat_9, cat_10, cat_11, cat_12, cat_13, cat_14, cat_15, cat_16, cat_17, cat_18, cat_19, cat_20, cat_21, cat_22, cat_23, cat_24, cat_25, emb_0, emb_1, emb_2, emb_3, emb_4, emb_5, emb_6, emb_7, emb_8, emb_9, emb_10, emb_11, emb_12, emb_13, emb_14, emb_15, emb_16, emb_17, emb_18, emb_19, emb_20, emb_21, emb_22, emb_23, emb_24, emb_25):
    raise NotImplementedError("write your pallas kernel here")


# speedup vs baseline: 9.9672x; 9.9672x over previous
import functools

import jax
import jax.numpy as jnp
from jax import lax
from jax.experimental import pallas as pl
from jax.experimental.pallas import tpu as pltpu
from jax.experimental.pallas import tpu_sc as plsc

NUM_COLS = 13
CAT_COLS = 26
VOCAB = 100
EMB_DIM = 64
BATCH = 16384

NC = 2
NS = 16
LANES = 16
NW = NC * NS
BPW = BATCH // NW
IDX_MINOR = 128
IDX_ROWS = BPW // IDX_MINOR
NUM_PAD = 16

_mesh = plsc.VectorSubcoreMesh(core_axis_name="c", subcore_axis_name="s")


@functools.partial(
    pl.kernel,
    mesh=_mesh,
    out_type=jax.ShapeDtypeStruct((BATCH, CAT_COLS * EMB_DIM), jnp.float32),
    scratch_types=[
        pltpu.VMEM_SHARED((CAT_COLS * VOCAB, 2 * EMB_DIM), jnp.float32),
        pltpu.VMEM_SHARED((CAT_COLS * VOCAB, 2 * EMB_DIM), jnp.float32),
        pltpu.VMEM((IDX_ROWS, IDX_MINOR), jnp.int32),
        pltpu.VMEM((BPW, 2 * EMB_DIM), jnp.float32),
        pltpu.SemaphoreType.DMA,
    ],
)
def _cat_gather(tl_hbm, tr_hbm, idx_hbm, out_hbm, tl_sh, tr_sh, idx_v,
                rows_v, sem):
    c = lax.axis_index("c")
    s = lax.axis_index("s")
    wid = s * NC + c
    b0 = wid * BPW

    @pl.when(s == 0)
    def _stage():
        pltpu.sync_copy(tl_hbm, tl_sh)
        pltpu.sync_copy(tr_hbm, tr_sh)

    plsc.subcore_barrier()

    def one_pair(j, carry):
        for p in range(2):
            i = 2 * j + p
            pltpu.sync_copy(idx_hbm.at[i, wid], idx_v)
            off = i * VOCAB
            for r in range(IDX_ROWS):
                for g in range(IDX_MINOR // LANES):
                    sl = (r, pl.ds(g * LANES, LANES))
                    idx_v[sl] = idx_v[sl] + off
            src = tl_sh if p == 0 else tr_sh
            for r in range(IDX_ROWS):
                pltpu.async_copy(
                    src.at[idx_v.at[r]],
                    rows_v.at[pl.ds(r * IDX_MINOR, IDX_MINOR)],
                    sem,
                    add=(p == 1),
                ).wait()
        pltpu.sync_copy(
            rows_v,
            out_hbm.at[pl.ds(b0, BPW), pl.ds(j * 2 * EMB_DIM, 2 * EMB_DIM)],
        )
        return carry

    lax.fori_loop(0, CAT_COLS // 2, one_pair, 0)


def _nums_body(n_ref, o_ref):
    eye = jnp.eye(NUM_PAD, dtype=jnp.float32)
    o_ref[...] = lax.dot_general(
        n_ref[...], eye, (((0,), (0,)), ((), ())),
        preferred_element_type=jnp.float32,
        precision=lax.Precision.HIGHEST,
    )


_nums_transpose = pl.pallas_call(
    _nums_body,
    out_shape=jax.ShapeDtypeStruct((BATCH, NUM_PAD), jnp.float32),
)


def kernel(num_0, num_1, num_2, num_3, num_4, num_5, num_6, num_7, num_8, num_9, num_10, num_11, num_12, cat_0, cat_1, cat_2, cat_3, cat_4, cat_5, cat_6, cat_7, cat_8, cat_9, cat_10, cat_11, cat_12, cat_13, cat_14, cat_15, cat_16, cat_17, cat_18, cat_19, cat_20, cat_21, cat_22, cat_23, cat_24, cat_25, emb_0, emb_1, emb_2, emb_3, emb_4, emb_5, emb_6, emb_7, emb_8, emb_9, emb_10, emb_11, emb_12, emb_13, emb_14, emb_15, emb_16, emb_17, emb_18, emb_19, emb_20, emb_21, emb_22, emb_23, emb_24, emb_25):
    nums = [num_0, num_1, num_2, num_3, num_4, num_5, num_6, num_7, num_8,
            num_9, num_10, num_11, num_12]
    cats = [cat_0, cat_1, cat_2, cat_3, cat_4, cat_5, cat_6, cat_7, cat_8,
            cat_9, cat_10, cat_11, cat_12, cat_13, cat_14, cat_15, cat_16,
            cat_17, cat_18, cat_19, cat_20, cat_21, cat_22, cat_23, cat_24,
            cat_25]
    embs = [emb_0, emb_1, emb_2, emb_3, emb_4, emb_5, emb_6, emb_7, emb_8,
            emb_9, emb_10, emb_11, emb_12, emb_13, emb_14, emb_15, emb_16,
            emb_17, emb_18, emb_19, emb_20, emb_21, emb_22, emb_23, emb_24,
            emb_25]

    table = jnp.concatenate(embs, axis=0)
    zeros = jnp.zeros_like(table)
    table_l = jnp.concatenate([table, zeros], axis=1)
    table_r = jnp.concatenate([zeros, table], axis=1)
    idx = jnp.stack(cats, axis=0).reshape(CAT_COLS, NW, IDX_ROWS, IDX_MINOR)
    x_cats = _cat_gather(table_l, table_r, idx)

    nums2d = jnp.concatenate(
        [jnp.stack(nums, axis=0),
         jnp.zeros((NUM_PAD - NUM_COLS, BATCH), jnp.float32)], axis=0)
    x_nums = _nums_transpose(nums2d)[:, :NUM_COLS]
    return (x_nums, x_cats)


# speedup vs baseline: 11.2938x; 1.1331x over previous
import functools

import jax
import jax.numpy as jnp
from jax import lax
from jax.experimental import pallas as pl
from jax.experimental.pallas import tpu as pltpu
from jax.experimental.pallas import tpu_sc as plsc

NUM_COLS = 13
CAT_COLS = 26
VOCAB = 100
EMB_DIM = 64
BATCH = 16384

NC = 2
NS = 16
LANES = 16
NW = NC * NS
BPW = BATCH // NW
IDX_MINOR = 128
IDX_ROWS = BPW // IDX_MINOR
HALF = BPW // 2
NUM_PAD = 16

_mesh = plsc.VectorSubcoreMesh(core_axis_name="c", subcore_axis_name="s")


@functools.partial(
    pl.kernel,
    mesh=_mesh,
    out_type=jax.ShapeDtypeStruct((BATCH, CAT_COLS * EMB_DIM), jnp.float32),
    scratch_types=[
        pltpu.VMEM_SHARED((CAT_COLS * VOCAB, 2 * EMB_DIM), jnp.float32),
        pltpu.VMEM_SHARED((CAT_COLS * VOCAB, 2 * EMB_DIM), jnp.float32),
        pltpu.VMEM((2, IDX_MINOR), jnp.int32),
        pltpu.VMEM((2, IDX_MINOR), jnp.int32),
        pltpu.VMEM((HALF, 2 * EMB_DIM), jnp.float32),
        pltpu.VMEM((HALF, 2 * EMB_DIM), jnp.float32),
        pltpu.SemaphoreType.DMA((2,)),
        pltpu.SemaphoreType.DMA((2,)),
    ],
)
def _cat_gather(tl_hbm, tr_hbm, idx_hbm, out_hbm, tl_sh, tr_sh, idxa_v,
                idxb_v, slab0, slab1, semg, semw):
    c = lax.axis_index("c")
    s = lax.axis_index("s")
    wid = s * NC + c
    b0 = wid * BPW
    slabs = (slab0, slab1)

    @pl.when(s == 0)
    def _stage():
        pltpu.sync_copy(tl_hbm, tl_sh)
        pltpu.sync_copy(tr_hbm, tr_sh)

    plsc.subcore_barrier()

    def one_item(t, carry):
        j = t // 2
        h = t % 2
        ia = 2 * j
        for ref, i in ((idxa_v, ia), (idxb_v, ia + 1)):
            pltpu.sync_copy(idx_hbm.at[i, wid, h], ref)
            off = i * VOCAB
            for r in range(2):
                for g in range(IDX_MINOR // LANES):
                    sl = (r, pl.ds(g * LANES, LANES))
                    ref[sl] = ref[sl] + off

        def run(slab):
            @pl.when(t >= 2)
            def _drain_write():
                pltpu.make_async_copy(
                    slab,
                    out_hbm.at[pl.ds(b0, HALF), pl.ds(0, 2 * EMB_DIM)],
                    semw.at[h],
                ).wait()

            for r in range(2):
                pltpu.async_copy(
                    tl_sh.at[idxa_v.at[r]],
                    slab.at[pl.ds(r * IDX_MINOR, IDX_MINOR)],
                    semg.at[r],
                )
            for r in range(2):
                pltpu.make_async_copy(
                    tl_sh.at[idxa_v.at[r]],
                    slab.at[pl.ds(r * IDX_MINOR, IDX_MINOR)],
                    semg.at[r],
                ).wait()
                pltpu.async_copy(
                    tr_sh.at[idxb_v.at[r]],
                    slab.at[pl.ds(r * IDX_MINOR, IDX_MINOR)],
                    semg.at[r],
                    add=True,
                )
            for r in range(2):
                pltpu.make_async_copy(
                    tr_sh.at[idxb_v.at[r]],
                    slab.at[pl.ds(r * IDX_MINOR, IDX_MINOR)],
                    semg.at[r],
                ).wait()
            pltpu.async_copy(
                slab,
                out_hbm.at[pl.ds(b0 + h * HALF, HALF),
                           pl.ds(j * 2 * EMB_DIM, 2 * EMB_DIM)],
                semw.at[h],
            )

        @pl.when(h == 0)
        def _even():
            run(slabs[0])

        @pl.when(h == 1)
        def _odd():
            run(slabs[1])

        return carry

    lax.fori_loop(0, CAT_COLS, one_item, 0)

    for h in range(2):
        pltpu.make_async_copy(
            slabs[h],
            out_hbm.at[pl.ds(b0, HALF), pl.ds(0, 2 * EMB_DIM)],
            semw.at[h],
        ).wait()


def _nums_body(n_ref, o_ref):
    eye = jnp.eye(NUM_PAD, dtype=jnp.float32)
    o_ref[...] = lax.dot_general(
        n_ref[...], eye, (((0,), (0,)), ((), ())),
        preferred_element_type=jnp.float32,
        precision=lax.Precision.HIGHEST,
    )


_nums_transpose = pl.pallas_call(
    _nums_body,
    out_shape=jax.ShapeDtypeStruct((BATCH, NUM_PAD), jnp.float32),
)


def kernel(num_0, num_1, num_2, num_3, num_4, num_5, num_6, num_7, num_8, num_9, num_10, num_11, num_12, cat_0, cat_1, cat_2, cat_3, cat_4, cat_5, cat_6, cat_7, cat_8, cat_9, cat_10, cat_11, cat_12, cat_13, cat_14, cat_15, cat_16, cat_17, cat_18, cat_19, cat_20, cat_21, cat_22, cat_23, cat_24, cat_25, emb_0, emb_1, emb_2, emb_3, emb_4, emb_5, emb_6, emb_7, emb_8, emb_9, emb_10, emb_11, emb_12, emb_13, emb_14, emb_15, emb_16, emb_17, emb_18, emb_19, emb_20, emb_21, emb_22, emb_23, emb_24, emb_25):
    nums = [num_0, num_1, num_2, num_3, num_4, num_5, num_6, num_7, num_8,
            num_9, num_10, num_11, num_12]
    cats = [cat_0, cat_1, cat_2, cat_3, cat_4, cat_5, cat_6, cat_7, cat_8,
            cat_9, cat_10, cat_11, cat_12, cat_13, cat_14, cat_15, cat_16,
            cat_17, cat_18, cat_19, cat_20, cat_21, cat_22, cat_23, cat_24,
            cat_25]
    embs = [emb_0, emb_1, emb_2, emb_3, emb_4, emb_5, emb_6, emb_7, emb_8,
            emb_9, emb_10, emb_11, emb_12, emb_13, emb_14, emb_15, emb_16,
            emb_17, emb_18, emb_19, emb_20, emb_21, emb_22, emb_23, emb_24,
            emb_25]

    table = jnp.concatenate(embs, axis=0)
    zeros = jnp.zeros_like(table)
    table_l = jnp.concatenate([table, zeros], axis=1)
    table_r = jnp.concatenate([zeros, table], axis=1)
    idx = jnp.stack(cats, axis=0).reshape(CAT_COLS, NW, 2, 2, IDX_MINOR)
    x_cats = _cat_gather(table_l, table_r, idx)

    nums2d = jnp.concatenate(
        [jnp.stack(nums, axis=0),
         jnp.zeros((NUM_PAD - NUM_COLS, BATCH), jnp.float32)], axis=0)
    x_nums = _nums_transpose(nums2d)[:, :NUM_COLS]
    return (x_nums, x_cats)
